# baseline (speedup 1.0000x reference)
import jax
import jax.numpy as jnp
from jax.experimental import pallas as pl


def _malconv_kernel(zr_ref, w1_ref, w2_ref, b1_ref, b2_ref, out_ref):
    zb = zr_ref[0]
    c1 = jnp.dot(zb, w1_ref[...], preferred_element_type=jnp.float32) + b1_ref[...]
    c2 = jnp.dot(zb, w2_ref[...], preferred_element_type=jnp.float32) + b2_ref[...]
    g = c1 * jax.nn.sigmoid(c2)
    out_ref[0] = jnp.max(g, axis=0, keepdims=True)


def kernel(z, W1, b1, W2, b2):
    B, T, E = z.shape
    C, _, K = W1.shape
    NW = T // K
    KE = K * E
    zr = z.reshape(B, NW, KE)
    W1t = W1.transpose(2, 1, 0).reshape(KE, C)
    W2t = W2.transpose(2, 1, 0).reshape(KE, C)
    out = pl.pallas_call(
        _malconv_kernel,
        grid=(B,),
        in_specs=[
            pl.BlockSpec((1, NW, KE), lambda b: (b, 0, 0)),
            pl.BlockSpec((KE, C), lambda b: (0, 0)),
            pl.BlockSpec((KE, C), lambda b: (0, 0)),
            pl.BlockSpec((1, C), lambda b: (0, 0)),
            pl.BlockSpec((1, C), lambda b: (0, 0)),
        ],
        out_specs=pl.BlockSpec((1, 1, C), lambda b: (b, 0, 0)),
        out_shape=jax.ShapeDtypeStruct((B, 1, C), jnp.float32),
    )(zr, W1t, W2t, b1.reshape(1, C), b2.reshape(1, C))
    return out.reshape(B, C)


# speedup vs baseline: 1.0017x; 1.0017x over previous
import jax
import jax.numpy as jnp
from jax.experimental import pallas as pl


def _malconv_kernel(zv_ref, w1_ref, w2_ref, b1_ref, b2_ref, out_ref):
    zb = zv_ref[0]
    nw = zb.shape[0]
    c1 = jnp.zeros((nw, w1_ref.shape[2]), dtype=jnp.float32)
    c2 = jnp.zeros((nw, w1_ref.shape[2]), dtype=jnp.float32)
    for a in range(32):
        za = zb[:, a, :]
        c1 = c1 + jnp.dot(za, w1_ref[a], preferred_element_type=jnp.float32)
        c2 = c2 + jnp.dot(za, w2_ref[a], preferred_element_type=jnp.float32)
    g = (c1 + b1_ref[...]) * jax.nn.sigmoid(c2 + b2_ref[...])
    out_ref[0] = jnp.max(g, axis=0, keepdims=True)


def _permute_weights(W):
    C, E, K = W.shape
    return W.reshape(C, E, K // 16, 16).transpose(2, 3, 1, 0).reshape(K // 16, 16 * E, C)


def kernel(z, W1, b1, W2, b2):
    B, T, E = z.shape
    C, _, K = W1.shape
    NW = T // K
    zv = z.reshape(B, NW, 32, 128)
    W1v = _permute_weights(W1)
    W2v = _permute_weights(W2)
    out = pl.pallas_call(
        _malconv_kernel,
        grid=(B,),
        in_specs=[
            pl.BlockSpec((1, NW, 32, 128), lambda b: (b, 0, 0, 0)),
            pl.BlockSpec((32, 128, C), lambda b: (0, 0, 0)),
            pl.BlockSpec((32, 128, C), lambda b: (0, 0, 0)),
            pl.BlockSpec((1, C), lambda b: (0, 0)),
            pl.BlockSpec((1, C), lambda b: (0, 0)),
        ],
        out_specs=pl.BlockSpec((1, 1, C), lambda b: (b, 0, 0)),
        out_shape=jax.ShapeDtypeStruct((B, 1, C), jnp.float32),
    )(zv, W1v, W2v, b1.reshape(1, C), b2.reshape(1, C))
    return out.reshape(B, C)


# speedup vs baseline: 14.5333x; 14.5080x over previous
import jax
import jax.numpy as jnp
from jax import lax
from jax.experimental import pallas as pl
from jax.experimental.pallas import tpu as pltpu


def _malconv_kernel(zt_ref, w1_ref, w2_ref, b1_ref, b2_ref, out_ref, w1s, w2s):
    zbt = zt_ref[0]
    e, tc = zbt.shape
    nw = tc // 512

    @pl.when(pl.program_id(0) == 0)
    def _():
        for ei in range(e):
            w1s[ei] = w1_ref[:, ei, :]
            w2s[ei] = w2_ref[:, ei, :]

    zz = zbt.reshape(e, nw, 512).transpose(1, 0, 2).reshape(nw, 512 * e)
    dn = (((1,), (1,)), ((), ()))
    c1 = jnp.zeros((nw, w1_ref.shape[0]), dtype=jnp.float32)
    c2 = jnp.zeros((nw, w1_ref.shape[0]), dtype=jnp.float32)
    for ei in range(e):
        zze = zz[:, ei * 512:(ei + 1) * 512]
        c1 = c1 + lax.dot_general(zze, w1s[ei], dn, preferred_element_type=jnp.float32)
        c2 = c2 + lax.dot_general(zze, w2s[ei], dn, preferred_element_type=jnp.float32)
    g = (c1 + b1_ref[...]) * jax.nn.sigmoid(c2 + b2_ref[...])
    out_ref[0] = jnp.max(g, axis=0, keepdims=True)


def kernel(z, W1, b1, W2, b2):
    B, T, E = z.shape
    C, _, K = W1.shape
    KE = K * E
    zt = jnp.transpose(z, (0, 2, 1))
    out = pl.pallas_call(
        _malconv_kernel,
        grid=(B,),
        in_specs=[
            pl.BlockSpec((1, E, T), lambda b: (b, 0, 0)),
            pl.BlockSpec((C, E, K), lambda b: (0, 0, 0)),
            pl.BlockSpec((C, E, K), lambda b: (0, 0, 0)),
            pl.BlockSpec((1, C), lambda b: (0, 0)),
            pl.BlockSpec((1, C), lambda b: (0, 0)),
        ],
        out_specs=pl.BlockSpec((1, 1, C), lambda b: (b, 0, 0)),
        out_shape=jax.ShapeDtypeStruct((B, 1, C), jnp.float32),
        scratch_shapes=[
            pltpu.VMEM((E, C, K), jnp.float32),
            pltpu.VMEM((E, C, K), jnp.float32),
        ],
    )(zt, W1, W2, b1.reshape(1, C), b2.reshape(1, C))
    return out.reshape(B, C)
